# baseline (speedup 1.0000x reference)
import functools

import jax
import jax.numpy as jnp
from jax import lax
from jax.experimental import pallas as pl
from jax.experimental.pallas import tpu as pltpu
from jax.experimental.pallas import tpu_sc as plsc

_N = 10000
_C = 128
_E = 320000
_NC = 2
_NS = 16
_NW = _NC * _NS
_NPAD = 10240
_RPT = _NPAD // _NS

_sc_mesh = plsc.VectorSubcoreMesh(core_axis_name="c", subcore_axis_name="s")


_KD = 128
_NCHUNK_D = 10240 // _KD


@functools.partial(
    pl.kernel,
    out_type=jax.ShapeDtypeStruct((_NC * _NPAD,), jnp.float32),
    mesh=_sc_mesh,
    scratch_types=[
        pltpu.VMEM((_NCHUNK_D, _KD), jnp.int32),
        pltpu.VMEM((_KD,), jnp.float32),
        pltpu.VMEM((_RPT,), jnp.float32),
        pltpu.VMEM_SHARED((_NPAD,), jnp.float32),
        pltpu.SemaphoreType.DMA,
    ],
)
def _deg_kernel(dst2_hbm, out_hbm, slab_v, ones_v, stage_v, acc_sh, sem):
    c = lax.axis_index("c")
    s = lax.axis_index("s")
    w = c * _NS + s
    rbase = s * _RPT

    def fill_ones(i, _):
        ones_v[pl.ds(i * 16, 16)] = jnp.ones((16,), jnp.float32)
        return 0

    lax.fori_loop(0, _KD // 16, fill_ones, 0)

    def fill_init(i, _):
        stage_v[pl.ds(i * 16, 16)] = jnp.ones((16,), jnp.float32)
        return 0

    lax.fori_loop(0, _RPT // 16, fill_init, 0)

    pltpu.sync_copy(dst2_hbm.at[pl.ds(w * _NCHUNK_D, _NCHUNK_D)], slab_v)
    pltpu.sync_copy(stage_v, acc_sh.at[pl.ds(rbase, _RPT)])
    plsc.subcore_barrier()

    def fire(i, _):
        pltpu.async_copy(ones_v, acc_sh.at[slab_v.at[i]], sem, add=True)
        return 0

    lax.fori_loop(0, _NCHUNK_D, fire, 0)

    def drain(i, _):
        pltpu.make_async_copy(ones_v, acc_sh.at[slab_v.at[i]], sem).wait()
        return 0

    lax.fori_loop(0, _NCHUNK_D, drain, 0)
    plsc.subcore_barrier()

    pltpu.sync_copy(acc_sh.at[pl.ds(rbase, _RPT)], out_hbm.at[pl.ds(c * _NPAD + rbase, _RPT)])


_KA = 128
_EPT_PAD = 10240
_EPAD = _EPT_PAD * _NW
_NCHUNK = _EPT_PAD // _KA


@functools.partial(
    pl.kernel,
    out_type=jax.ShapeDtypeStruct((_NC, _NPAD, _C), jnp.float32),
    mesh=_sc_mesh,
    scratch_types=[
        pltpu.VMEM((_NCHUNK, _KA), jnp.int32),
        pltpu.VMEM((_KA,), jnp.int32),
        pltpu.VMEM((_KA,), jnp.int32),
        pltpu.VMEM((_KA,), jnp.int32),
        pltpu.VMEM((_KA,), jnp.int32),
        pltpu.VMEM((_KA, _C), jnp.float32),
        pltpu.VMEM((_KA, _C), jnp.float32),
        pltpu.VMEM_SHARED((_NPAD, _C), jnp.float32),
        pltpu.SemaphoreType.DMA,
        pltpu.SemaphoreType.DMA,
        pltpu.SemaphoreType.DMA,
        pltpu.SemaphoreType.DMA,
    ],
)
def _agg_kernel(packed_hbm, y_hbm, out_hbm, slab_v, src_a, dst_a, src_b, dst_b,
                rows_a, rows_b, acc_sh, gsem_a, gsem_b, ssem_a, ssem_b):
    c = lax.axis_index("c")
    s = lax.axis_index("s")
    w = c * _NS + s
    rbase = s * _RPT

    pltpu.async_copy(y_hbm.at[pl.ds(rbase, _RPT)], acc_sh.at[pl.ds(rbase, _RPT)], ssem_b)
    pltpu.async_copy(packed_hbm.at[pl.ds(w * _NCHUNK, _NCHUNK)], slab_v, gsem_b)
    pltpu.make_async_copy(packed_hbm.at[pl.ds(w * _NCHUNK, _NCHUNK)], slab_v, gsem_b).wait()

    def unpack(i, src_v, dst_v):
        for r in range(_KA // 16):
            v = slab_v[i, pl.ds(r * 16, 16)]
            src_v[pl.ds(r * 16, 16)] = v & 0xFFFF
            dst_v[pl.ds(r * 16, 16)] = v >> 16

    def gather(src_v, rows_v, gsem):
        pltpu.async_copy(y_hbm.at[src_v], rows_v, gsem)

    def scatter(rows_v, dst_v, ssem):
        pltpu.async_copy(rows_v, acc_sh.at[dst_v], ssem, add=True)

    def wait_gather(src_v, rows_v, gsem):
        pltpu.make_async_copy(y_hbm.at[src_v], rows_v, gsem).wait()

    def wait_scatter(rows_v, dst_v, ssem):
        pltpu.make_async_copy(rows_v, acc_sh.at[dst_v], ssem).wait()

    unpack(0, src_a, dst_a)
    gather(src_a, rows_a, gsem_a)
    unpack(1, src_b, dst_b)
    gather(src_b, rows_b, gsem_b)
    pltpu.make_async_copy(y_hbm.at[pl.ds(rbase, _RPT)], acc_sh.at[pl.ds(rbase, _RPT)], ssem_b).wait()
    plsc.subcore_barrier()
    wait_gather(src_a, rows_a, gsem_a)
    scatter(rows_a, dst_a, ssem_a)

    def pair(j, _):
        wait_scatter(rows_a, dst_a, ssem_a)
        unpack(2 * j + 2, src_a, dst_a)
        wait_gather(src_b, rows_b, gsem_b)
        scatter(rows_b, dst_b, ssem_b)
        gather(src_a, rows_a, gsem_a)
        wait_scatter(rows_b, dst_b, ssem_b)
        unpack(2 * j + 3, src_b, dst_b)
        wait_gather(src_a, rows_a, gsem_a)
        scatter(rows_a, dst_a, ssem_a)
        gather(src_b, rows_b, gsem_b)
        return 0

    lax.fori_loop(0, (_NCHUNK - 2) // 2, pair, 0)

    wait_scatter(rows_a, dst_a, ssem_a)
    wait_gather(src_b, rows_b, gsem_b)
    scatter(rows_b, dst_b, ssem_b)
    wait_scatter(rows_b, dst_b, ssem_b)
    plsc.subcore_barrier()

    pltpu.sync_copy(acc_sh.at[pl.ds(rbase, _RPT)], out_hbm.at[c, pl.ds(rbase, _RPT)])


_ROWS_BLK = 2048


def _enc_body(x_ref, wt_ref, b_ref, y_ref):
    xw = jnp.dot(x_ref[...], wt_ref[...], preferred_element_type=jnp.float32)
    y_ref[...] = jnp.maximum(xw + b_ref[...], 0.0)


def _out_body(s0_ref, s1_ref, y_ref, d0_ref, d1_ref, wzt_ref, wlzat_ref,
              bz_ref, blz_ref, wht_ref, wlhat_ref, bh_ref, blh_ref, o_ref):
    dinv = lax.rsqrt(d0_ref[...] + d1_ref[...] - 1.0)
    agg = (s0_ref[...] + s1_ref[...] - y_ref[...]) * dinv
    wze = jnp.dot(wzt_ref[...], wlzat_ref[...], preferred_element_type=jnp.float32)
    bze = jnp.dot(bz_ref[...], wlzat_ref[...], preferred_element_type=jnp.float32) + blz_ref[...]
    whe = jnp.dot(wht_ref[...], wlhat_ref[...], preferred_element_type=jnp.float32)
    bhe = jnp.dot(bh_ref[...], wlhat_ref[...], preferred_element_type=jnp.float32) + blh_ref[...]
    z = jax.nn.sigmoid(jnp.dot(agg, wze, preferred_element_type=jnp.float32) + bze)
    ht = jnp.tanh(jnp.dot(agg, whe, preferred_element_type=jnp.float32) + bhe)
    o_ref[...] = (1.0 - z) * ht


def kernel(x, edge_index, edge_attr, return_embedding, W_node, b_node, W_edge,
           b_edge, Wz, bz, Wlz, blz, Wr, br, Wlr, blr, Wh, bh, Wlh, blh):
    src = edge_index[0]
    dst = edge_index[1]

    pad = _EPAD - _E
    pad_iota = jnp.arange(pad, dtype=jnp.int32)
    src_pad = jnp.concatenate([src, pad_iota % _N])
    dst_pad = jnp.concatenate([dst, _N + pad_iota % (_NPAD - _N)])
    packed = (src_pad | (dst_pad << 16)).reshape(_NW * _NCHUNK, _KA)

    deg_flat = _deg_kernel(dst_pad.reshape(_NW * _NCHUNK_D, _KD))
    d0 = deg_flat[:_NPAD].reshape(_NPAD, 1)
    d1 = deg_flat[_NPAD:].reshape(_NPAD, 1)

    grid = _NPAD // _ROWS_BLK
    rows_spec = pl.BlockSpec((_ROWS_BLK, _C), lambda i: (i, 0))
    col_spec = pl.BlockSpec((_ROWS_BLK, 1), lambda i: (i, 0))
    w_spec = pl.BlockSpec((_C, _C), lambda i: (0, 0))
    b_spec = pl.BlockSpec((1, _C), lambda i: (0, 0))

    x_enc = pl.pallas_call(
        _enc_body,
        grid=(grid,),
        in_specs=[rows_spec, w_spec, b_spec],
        out_specs=rows_spec,
        out_shape=jax.ShapeDtypeStruct((_NPAD, _C), jnp.float32),
    )(x, W_node.T, b_node.reshape(1, _C))
    y = lax.rsqrt(d0 + d1 - 1.0) * x_enc

    s_parts = _agg_kernel(packed, y)

    out_blk = _N // 2
    orow_spec = pl.BlockSpec((out_blk, _C), lambda i: (i, 0))
    ocol_spec = pl.BlockSpec((out_blk, 1), lambda i: (i, 0))
    ow_spec = pl.BlockSpec((_C, _C), lambda i: (0, 0))
    ob_spec = pl.BlockSpec((1, _C), lambda i: (0, 0))
    out = pl.pallas_call(
        _out_body,
        grid=(2,),
        in_specs=[orow_spec, orow_spec, orow_spec, ocol_spec, ocol_spec,
                  ow_spec, ow_spec, ob_spec, ob_spec, ow_spec, ow_spec, ob_spec, ob_spec],
        out_specs=orow_spec,
        out_shape=jax.ShapeDtypeStruct((_N, _C), jnp.float32),
    )(s_parts[0], s_parts[1], y, d0, d1,
      Wz.T, Wlz[:, :_C].T, bz.reshape(1, _C), blz.reshape(1, _C),
      Wh.T, Wlh[:, :_C].T, bh.reshape(1, _C), blh.reshape(1, _C))

    return out


# speedup vs baseline: 1.0083x; 1.0083x over previous
import functools

import jax
import jax.numpy as jnp
from jax import lax
from jax.experimental import pallas as pl
from jax.experimental.pallas import tpu as pltpu
from jax.experimental.pallas import tpu_sc as plsc

_N = 10000
_C = 128
_E = 320000
_NC = 2
_NS = 16
_NW = _NC * _NS
_NPAD = 10240
_RPT = _NPAD // _NS

_sc_mesh = plsc.VectorSubcoreMesh(core_axis_name="c", subcore_axis_name="s")


_KD = 128
_NCHUNK_D = 10240 // _KD


@functools.partial(
    pl.kernel,
    out_type=jax.ShapeDtypeStruct((_NC * _NPAD,), jnp.float32),
    mesh=_sc_mesh,
    scratch_types=[
        pltpu.VMEM((_NCHUNK_D, _KD), jnp.int32),
        pltpu.VMEM((_KD,), jnp.float32),
        pltpu.VMEM((_RPT,), jnp.float32),
        pltpu.VMEM_SHARED((_NPAD,), jnp.float32),
        pltpu.SemaphoreType.DMA,
    ],
)
def _deg_kernel(dst2_hbm, out_hbm, slab_v, ones_v, stage_v, acc_sh, sem):
    c = lax.axis_index("c")
    s = lax.axis_index("s")
    w = c * _NS + s
    rbase = s * _RPT

    def fill_ones(i, _):
        ones_v[pl.ds(i * 16, 16)] = jnp.ones((16,), jnp.float32)
        return 0

    lax.fori_loop(0, _KD // 16, fill_ones, 0)

    def fill_init(i, _):
        stage_v[pl.ds(i * 16, 16)] = jnp.ones((16,), jnp.float32)
        return 0

    lax.fori_loop(0, _RPT // 16, fill_init, 0)

    pltpu.sync_copy(dst2_hbm.at[pl.ds(w * _NCHUNK_D, _NCHUNK_D)], slab_v)
    pltpu.sync_copy(stage_v, acc_sh.at[pl.ds(rbase, _RPT)])
    plsc.subcore_barrier()

    def fire(i, _):
        pltpu.async_copy(ones_v, acc_sh.at[slab_v.at[i]], sem, add=True)
        return 0

    lax.fori_loop(0, _NCHUNK_D, fire, 0)

    def drain(i, _):
        pltpu.make_async_copy(ones_v, acc_sh.at[slab_v.at[i]], sem).wait()
        return 0

    lax.fori_loop(0, _NCHUNK_D, drain, 0)
    plsc.subcore_barrier()

    pltpu.sync_copy(acc_sh.at[pl.ds(rbase, _RPT)], out_hbm.at[pl.ds(c * _NPAD + rbase, _RPT)])


_KA = 128
_EPT_PAD = 10240
_EPAD = _EPT_PAD * _NW
_NCHUNK = _EPT_PAD // _KA


@functools.partial(
    pl.kernel,
    out_type=jax.ShapeDtypeStruct((_NC, _NPAD, _C), jnp.float32),
    mesh=_sc_mesh,
    scratch_types=[
        pltpu.VMEM((_NCHUNK, _KA), jnp.int32),
        pltpu.VMEM((_KA,), jnp.int32),
        pltpu.VMEM((_KA,), jnp.int32),
        pltpu.VMEM((_KA,), jnp.int32),
        pltpu.VMEM((_KA,), jnp.int32),
        pltpu.VMEM((_KA, _C), jnp.float32),
        pltpu.VMEM((_KA, _C), jnp.float32),
        pltpu.VMEM_SHARED((_NPAD, _C), jnp.float32),
        pltpu.SemaphoreType.DMA,
        pltpu.SemaphoreType.DMA,
        pltpu.SemaphoreType.DMA,
        pltpu.SemaphoreType.DMA,
    ],
)
def _agg_kernel(packed_hbm, y_hbm, out_hbm, slab_v, src_a, dst_a, src_b, dst_b,
                rows_a, rows_b, acc_sh, gsem_a, gsem_b, ssem_a, ssem_b):
    c = lax.axis_index("c")
    s = lax.axis_index("s")
    w = c * _NS + s
    rbase = s * _RPT

    pltpu.async_copy(y_hbm.at[pl.ds(rbase, _RPT)], acc_sh.at[pl.ds(rbase, _RPT)], ssem_b)
    pltpu.async_copy(packed_hbm.at[pl.ds(w * _NCHUNK, _NCHUNK)], slab_v, gsem_b)
    pltpu.make_async_copy(packed_hbm.at[pl.ds(w * _NCHUNK, _NCHUNK)], slab_v, gsem_b).wait()

    def unpack(i, src_v, dst_v):
        for r in range(_KA // 16):
            v = slab_v[i, pl.ds(r * 16, 16)]
            src_v[pl.ds(r * 16, 16)] = v & 0xFFFF
            dst_v[pl.ds(r * 16, 16)] = v >> 16

    def gather(src_v, rows_v, gsem):
        pltpu.async_copy(y_hbm.at[src_v], rows_v, gsem)

    def scatter(rows_v, dst_v, ssem):
        pltpu.async_copy(rows_v, acc_sh.at[dst_v], ssem, add=True)

    def wait_gather(src_v, rows_v, gsem):
        pltpu.make_async_copy(y_hbm.at[src_v], rows_v, gsem).wait()

    def wait_scatter(rows_v, dst_v, ssem):
        pltpu.make_async_copy(rows_v, acc_sh.at[dst_v], ssem).wait()

    unpack(0, src_a, dst_a)
    gather(src_a, rows_a, gsem_a)
    unpack(1, src_b, dst_b)
    gather(src_b, rows_b, gsem_b)
    pltpu.make_async_copy(y_hbm.at[pl.ds(rbase, _RPT)], acc_sh.at[pl.ds(rbase, _RPT)], ssem_b).wait()
    plsc.subcore_barrier()
    wait_gather(src_a, rows_a, gsem_a)
    scatter(rows_a, dst_a, ssem_a)

    def pair(j, _):
        wait_scatter(rows_a, dst_a, ssem_a)
        unpack(2 * j + 2, src_a, dst_a)
        wait_gather(src_b, rows_b, gsem_b)
        scatter(rows_b, dst_b, ssem_b)
        gather(src_a, rows_a, gsem_a)
        wait_scatter(rows_b, dst_b, ssem_b)
        unpack(2 * j + 3, src_b, dst_b)
        wait_gather(src_a, rows_a, gsem_a)
        scatter(rows_a, dst_a, ssem_a)
        gather(src_b, rows_b, gsem_b)
        return 0

    lax.fori_loop(0, (_NCHUNK - 2) // 2, pair, 0)

    wait_scatter(rows_a, dst_a, ssem_a)
    wait_gather(src_b, rows_b, gsem_b)
    scatter(rows_b, dst_b, ssem_b)
    wait_scatter(rows_b, dst_b, ssem_b)
    plsc.subcore_barrier()

    pltpu.sync_copy(acc_sh.at[pl.ds(rbase, _RPT)], out_hbm.at[c, pl.ds(rbase, _RPT)])


_ROWS_BLK = 2048


def _enc_body(x_ref, wt_ref, b_ref, y_ref):
    xw = jnp.dot(x_ref[...], wt_ref[...], preferred_element_type=jnp.float32)
    y_ref[...] = jnp.maximum(xw + b_ref[...], 0.0)


def _out_body(s0_ref, s1_ref, y_ref, d0_ref, d1_ref, wzt_ref, wlzat_ref,
              bz_ref, blz_ref, wht_ref, wlhat_ref, bh_ref, blh_ref, o_ref):
    dinv = lax.rsqrt(d0_ref[...] + d1_ref[...] - 1.0)
    agg = (s0_ref[...] + s1_ref[...] - y_ref[...]) * dinv
    wze = jnp.dot(wzt_ref[...], wlzat_ref[...], preferred_element_type=jnp.float32)
    bze = jnp.dot(bz_ref[...], wlzat_ref[...], preferred_element_type=jnp.float32) + blz_ref[...]
    whe = jnp.dot(wht_ref[...], wlhat_ref[...], preferred_element_type=jnp.float32)
    bhe = jnp.dot(bh_ref[...], wlhat_ref[...], preferred_element_type=jnp.float32) + blh_ref[...]
    z = jax.nn.sigmoid(jnp.dot(agg, wze, preferred_element_type=jnp.float32) + bze)
    ht = jnp.tanh(jnp.dot(agg, whe, preferred_element_type=jnp.float32) + bhe)
    o_ref[...] = (1.0 - z) * ht


def kernel(x, edge_index, edge_attr, return_embedding, W_node, b_node, W_edge,
           b_edge, Wz, bz, Wlz, blz, Wr, br, Wlr, blr, Wh, bh, Wlh, blh):
    src = edge_index[0]
    dst = edge_index[1]

    pad = _EPAD - _E
    pad_iota = jnp.arange(pad, dtype=jnp.int32)
    src_pad = jnp.concatenate([src, pad_iota % _N])
    dst_pad = jnp.concatenate([dst, _N + pad_iota % (_NPAD - _N)])
    packed = (src_pad | (dst_pad << 16)).reshape(_NW * _NCHUNK, _KA)

    deg_flat = _deg_kernel(dst_pad.reshape(_NW * _NCHUNK_D, _KD))
    d0 = deg_flat[:_NPAD].reshape(_NPAD, 1)
    d1 = deg_flat[_NPAD:].reshape(_NPAD, 1)

    grid = _NPAD // _ROWS_BLK
    rows_spec = pl.BlockSpec((_ROWS_BLK, _C), lambda i: (i, 0))
    col_spec = pl.BlockSpec((_ROWS_BLK, 1), lambda i: (i, 0))
    w_spec = pl.BlockSpec((_C, _C), lambda i: (0, 0))
    b_spec = pl.BlockSpec((1, _C), lambda i: (0, 0))

    x_enc = pl.pallas_call(
        _enc_body,
        grid=(grid,),
        in_specs=[rows_spec, w_spec, b_spec],
        out_specs=rows_spec,
        out_shape=jax.ShapeDtypeStruct((_NPAD, _C), jnp.float32),
    )(x, W_node.T, b_node.reshape(1, _C))
    y = lax.rsqrt(d0 + d1 - 1.0) * x_enc

    s_parts = _agg_kernel(packed, y)

    out_blk = _N // 5
    orow_spec = pl.BlockSpec((out_blk, _C), lambda i: (i, 0))
    ocol_spec = pl.BlockSpec((out_blk, 1), lambda i: (i, 0))
    ow_spec = pl.BlockSpec((_C, _C), lambda i: (0, 0))
    ob_spec = pl.BlockSpec((1, _C), lambda i: (0, 0))
    out = pl.pallas_call(
        _out_body,
        grid=(5,),
        in_specs=[orow_spec, orow_spec, orow_spec, ocol_spec, ocol_spec,
                  ow_spec, ow_spec, ob_spec, ob_spec, ow_spec, ow_spec, ob_spec, ob_spec],
        out_specs=orow_spec,
        out_shape=jax.ShapeDtypeStruct((_N, _C), jnp.float32),
    )(s_parts[0], s_parts[1], y, d0, d1,
      Wz.T, Wlz[:, :_C].T, bz.reshape(1, _C), blz.reshape(1, _C),
      Wh.T, Wlh[:, :_C].T, bh.reshape(1, _C), blh.reshape(1, _C))

    return out
